# baseline (speedup 1.0000x reference)
import functools

import numpy as np

import jax
import jax.numpy as jnp
from jax import lax
from jax.experimental import pallas as pl
from jax.experimental.pallas import tpu as pltpu
from jax.experimental.pallas import tpu_sc as plsc

B = 16384
E = 128
L = 20
CD = 32

NC = 2
NS = 16
NW = NC * NS
BPW = B // NW
RC = 128
NCH = BPW * L // RC
NBUF = 2
LOOK = 1
CCH = 128
NCC = BPW // CCH

_sc_mesh = plsc.VectorSubcoreMesh(core_axis_name="c", subcore_axis_name="s")


@functools.partial(
    pl.kernel,
    out_type=(
        jax.ShapeDtypeStruct((B, E), jnp.float32),
        jax.ShapeDtypeStruct((B, E), jnp.float32),
        jax.ShapeDtypeStruct((B, E), jnp.float32),
    ),
    mesh=_sc_mesh,
    scratch_types=[
        pltpu.VMEM((NCH, RC), jnp.int32),
        pltpu.VMEM((NBUF, RC, E), jnp.float32),
        pltpu.VMEM((NCH, RC), jnp.int32),
        pltpu.VMEM((NCC, CCH), jnp.int32),
        pltpu.VMEM_SHARED((NS * BPW, E), jnp.float32),
        pltpu.SemaphoreType.DMA,
        pltpu.SemaphoreType.DMA((NBUF,)),
        pltpu.SemaphoreType.DMA((NBUF,)),
    ],
)
def _sc_gather(tidx_hbm, tseg_hbm, cidx_hbm, sidx_hbm, wemb_hbm, cemb_hbm, semb_hbm,
               ts_out, cv_out, sv_out,
               idx_v, rows_v, tgt_v, cidx_v, acc_sh, sem, gsems, ssems):
    cid = lax.axis_index("c")
    sid = lax.axis_index("s")
    wid = sid * NC + cid
    base = wid * BPW
    sbase = sid * BPW

    pltpu.sync_copy(tidx_hbm.at[wid], idx_v)
    pltpu.sync_copy(tseg_hbm.at[sid], tgt_v)

    def zero_row(r, _):
        for g in range(E // 16):
            rows_v[0, r, pl.ds(g * 16, 16)] = jnp.zeros((16,), jnp.float32)
        return 0

    lax.fori_loop(0, CCH, zero_row, 0)
    for k in range(BPW // CCH):
        pltpu.sync_copy(rows_v.at[0], acc_sh.at[pl.ds(sbase + k * CCH, CCH)])

    pltpu.sync_copy(cidx_hbm.at[wid], cidx_v)

    def cat_chunk(j, _):
        pltpu.async_copy(cemb_hbm.at[cidx_v.at[j]], rows_v.at[0], sem).wait()
        pltpu.sync_copy(rows_v.at[0], cv_out.at[pl.ds(base + j * CCH, CCH)])
        return 0

    lax.fori_loop(0, NCC, cat_chunk, 0)

    pltpu.sync_copy(sidx_hbm.at[wid], cidx_v)

    def subcat_chunk(j, _):
        pltpu.async_copy(semb_hbm.at[cidx_v.at[j]], rows_v.at[0], sem).wait()
        pltpu.sync_copy(rows_v.at[0], sv_out.at[pl.ds(base + j * CCH, CCH)])
        return 0

    lax.fori_loop(0, NCC, subcat_chunk, 0)

    for k in range(LOOK):
        pltpu.async_copy(wemb_hbm.at[idx_v.at[k]], rows_v.at[k], gsems.at[k])

    def title_chunk(j, _):
        for b in range(NBUF):
            @pl.when(j % NBUF == b)
            def _():
                pltpu.make_async_copy(
                    wemb_hbm.at[idx_v.at[0]], rows_v.at[b], gsems.at[b]
                ).wait()
                pltpu.async_copy(
                    rows_v.at[b], acc_sh.at[tgt_v.at[j]], ssems.at[b], add=True
                )

        nxt = j + LOOK

        @pl.when(nxt < NCH)
        def _():
            for b2 in range(NBUF):
                @pl.when(nxt % NBUF == b2)
                def _():
                    @pl.when(j >= LOOK)
                    def _():
                        pltpu.make_async_copy(
                            rows_v.at[b2], acc_sh.at[tgt_v.at[0]], ssems.at[b2]
                        ).wait()
                    pltpu.async_copy(
                        wemb_hbm.at[idx_v.at[nxt]], rows_v.at[b2], gsems.at[b2]
                    )

        return 0

    lax.fori_loop(0, NCH, title_chunk, 0)
    for k in range(NCH - LOOK, NCH):
        b = k % NBUF
        pltpu.make_async_copy(
            rows_v.at[b], acc_sh.at[tgt_v.at[0]], ssems.at[b]
        ).wait()
    pltpu.sync_copy(acc_sh.at[pl.ds(sbase, BPW)], ts_out.at[pl.ds(base, BPW)])


_BB = 2048


def _dense_body(ts_ref, cv_ref, sv_ref, wt_ref, bt_ref,
                wf1_ref, wf2_ref, wf3_ref, bf_ref, o_ref):
    tv = jnp.dot(ts_ref[...], wt_ref[...], preferred_element_type=jnp.float32)
    tv = jnp.maximum(tv + bt_ref[...], 0.0)
    acc = jnp.dot(tv, wf1_ref[...], preferred_element_type=jnp.float32)
    acc = acc + jnp.dot(cv_ref[...], wf2_ref[...], preferred_element_type=jnp.float32)
    acc = acc + jnp.dot(sv_ref[...], wf3_ref[...], preferred_element_type=jnp.float32)
    o_ref[...] = jnp.maximum(acc + bf_ref[...], 0.0)


_dense = pl.pallas_call(
    _dense_body,
    grid=(B // _BB,),
    in_specs=[
        pl.BlockSpec((_BB, E), lambda i: (i, 0)),
        pl.BlockSpec((_BB, E), lambda i: (i, 0)),
        pl.BlockSpec((_BB, E), lambda i: (i, 0)),
        pl.BlockSpec((E, CD), lambda i: (0, 0)),
        pl.BlockSpec((1, CD), lambda i: (0, 0)),
        pl.BlockSpec((CD, E), lambda i: (0, 0)),
        pl.BlockSpec((E, E), lambda i: (0, 0)),
        pl.BlockSpec((E, E), lambda i: (0, 0)),
        pl.BlockSpec((1, E), lambda i: (0, 0)),
    ],
    out_specs=pl.BlockSpec((_BB, E), lambda i: (i, 0)),
    out_shape=jax.ShapeDtypeStruct((B, E), jnp.float32),
)


def kernel(title, category, subcategory, word_emb, cat_emb, subcat_emb,
           W_title, b_title, W_final, b_final):
    tidx = title.astype(jnp.int32).reshape(NW, NCH, RC)
    tseg = jnp.asarray(
        (np.arange(NS) * BPW)[:, None, None]
        + (np.arange(NCH * RC) // L).reshape(NCH, RC)[None],
        dtype=jnp.int32,
    )
    cidx = category.astype(jnp.int32).reshape(NW, NCC, CCH)
    sidx = subcategory.astype(jnp.int32).reshape(NW, NCC, CCH)
    pad = ((0, 0), (0, E - CD))
    cemb_p = jnp.pad(cat_emb, pad)
    semb_p = jnp.pad(subcat_emb, pad)
    ts, cv, sv = _sc_gather(tidx, tseg, cidx, sidx, word_emb, cemb_p, semb_p)
    wpad = ((0, E - CD), (0, 0))
    return _dense(
        ts, cv, sv,
        W_title * jnp.float32(1.0 / L),
        b_title.reshape(1, CD),
        W_final[:CD],
        jnp.pad(W_final[CD:2 * CD], wpad),
        jnp.pad(W_final[2 * CD:], wpad),
        b_final.reshape(1, E),
    )


# speedup vs baseline: 1.1067x; 1.1067x over previous
import functools

import numpy as np

import jax
import jax.numpy as jnp
from jax import lax
from jax.experimental import pallas as pl
from jax.experimental.pallas import tpu as pltpu
from jax.experimental.pallas import tpu_sc as plsc

B = 16384
E = 128
L = 20
CD = 32

NC = 2
NS = 16
NW = NC * NS
BPW = B // NW
RC = 128
NCH = BPW * L // RC
NBUF = 4
CCH = 128
NCC = BPW // CCH
WB = 32
DRN = WB * L // RC
NBLK = BPW // WB

_sc_mesh = plsc.VectorSubcoreMesh(core_axis_name="c", subcore_axis_name="s")


@functools.partial(
    pl.kernel,
    out_type=(
        jax.ShapeDtypeStruct((B, E), jnp.float32),
        jax.ShapeDtypeStruct((B, E), jnp.float32),
        jax.ShapeDtypeStruct((B, E), jnp.float32),
    ),
    mesh=_sc_mesh,
    scratch_types=[
        pltpu.VMEM((NCH, RC), jnp.int32),
        pltpu.VMEM((NBUF, RC, E), jnp.float32),
        pltpu.VMEM((NCH, RC), jnp.int32),
        pltpu.VMEM((NCC, CCH), jnp.int32),
        pltpu.VMEM((WB, E), jnp.float32),
        pltpu.VMEM_SHARED((NS * 2 * WB, E), jnp.float32),
        pltpu.SemaphoreType.DMA,
        pltpu.SemaphoreType.DMA((NBUF,)),
    ],
)
def _sc_gather(tidx_hbm, tseg_hbm, cidx_hbm, sidx_hbm, wemb_hbm, cemb_hbm, semb_hbm,
               ts_out, cv_out, sv_out,
               idx_v, rows_v, tgt_v, cidx_v, zero_v, acc_sh, sem, gsems):
    cid = lax.axis_index("c")
    sid = lax.axis_index("s")
    wid = sid * NC + cid
    base = wid * BPW
    sbase = sid * 2 * WB

    pltpu.sync_copy(tidx_hbm.at[wid], idx_v)
    pltpu.sync_copy(tseg_hbm.at[sid], tgt_v)

    def zero_row(r, _):
        for g in range(E // 16):
            zero_v[r, pl.ds(g * 16, 16)] = jnp.zeros((16,), jnp.float32)
        return 0

    lax.fori_loop(0, WB, zero_row, 0)
    for k in range(2):
        pltpu.sync_copy(zero_v, acc_sh.at[pl.ds(sbase + k * WB, WB)])

    pltpu.sync_copy(cidx_hbm.at[wid], cidx_v)

    def cat_chunk(j, _):
        pltpu.async_copy(cemb_hbm.at[cidx_v.at[j]], rows_v.at[0], sem).wait()
        pltpu.sync_copy(rows_v.at[0], cv_out.at[pl.ds(base + j * CCH, CCH)])
        return 0

    lax.fori_loop(0, NCC, cat_chunk, 0)

    pltpu.sync_copy(sidx_hbm.at[wid], cidx_v)

    def subcat_chunk(j, _):
        pltpu.async_copy(semb_hbm.at[cidx_v.at[j]], rows_v.at[0], sem).wait()
        pltpu.sync_copy(rows_v.at[0], sv_out.at[pl.ds(base + j * CCH, CCH)])
        return 0

    lax.fori_loop(0, NCC, subcat_chunk, 0)

    for b in range(NBUF):
        pltpu.async_copy(wemb_hbm.at[idx_v.at[b]], rows_v.at[b], gsems.at[b])

    def title_chunk(j, _):
        for b in range(NBUF):
            @pl.when(j % NBUF == b)
            def _():
                pltpu.make_async_copy(
                    wemb_hbm.at[idx_v.at[0]], rows_v.at[b], gsems.at[b]
                ).wait()
                pltpu.sync_copy(rows_v.at[b], acc_sh.at[tgt_v.at[j]], add=True)
                nxt = j + NBUF

                @pl.when(nxt < NCH)
                def _():
                    pltpu.async_copy(
                        wemb_hbm.at[idx_v.at[nxt]], rows_v.at[b], gsems.at[b]
                    )

        @pl.when(j % DRN == DRN - 1)
        def _():
            d = j // DRN
            for p in range(2):
                @pl.when(d % 2 == p)
                def _():
                    wrow = sbase + p * WB
                    pltpu.sync_copy(acc_sh.at[pl.ds(wrow, WB)],
                                    ts_out.at[pl.ds(base + d * WB, WB)])
                    pltpu.sync_copy(zero_v, acc_sh.at[pl.ds(wrow, WB)])

        return 0

    lax.fori_loop(0, NCH, title_chunk, 0)


_BB = 2048


def _dense_body(ts_ref, cv_ref, sv_ref, wt_ref, bt_ref,
                wf1_ref, wf2_ref, wf3_ref, bf_ref, o_ref):
    tv = jnp.dot(ts_ref[...], wt_ref[...], preferred_element_type=jnp.float32)
    tv = jnp.maximum(tv + bt_ref[...], 0.0)
    acc = jnp.dot(tv, wf1_ref[...], preferred_element_type=jnp.float32)
    acc = acc + jnp.dot(cv_ref[...], wf2_ref[...], preferred_element_type=jnp.float32)
    acc = acc + jnp.dot(sv_ref[...], wf3_ref[...], preferred_element_type=jnp.float32)
    o_ref[...] = jnp.maximum(acc + bf_ref[...], 0.0)


_dense = pl.pallas_call(
    _dense_body,
    grid=(B // _BB,),
    in_specs=[
        pl.BlockSpec((_BB, E), lambda i: (i, 0)),
        pl.BlockSpec((_BB, E), lambda i: (i, 0)),
        pl.BlockSpec((_BB, E), lambda i: (i, 0)),
        pl.BlockSpec((E, CD), lambda i: (0, 0)),
        pl.BlockSpec((1, CD), lambda i: (0, 0)),
        pl.BlockSpec((CD, E), lambda i: (0, 0)),
        pl.BlockSpec((E, E), lambda i: (0, 0)),
        pl.BlockSpec((E, E), lambda i: (0, 0)),
        pl.BlockSpec((1, E), lambda i: (0, 0)),
    ],
    out_specs=pl.BlockSpec((_BB, E), lambda i: (i, 0)),
    out_shape=jax.ShapeDtypeStruct((B, E), jnp.float32),
)


def kernel(title, category, subcategory, word_emb, cat_emb, subcat_emb,
           W_title, b_title, W_final, b_final):
    tidx = title.astype(jnp.int32).reshape(NW, NCH, RC)
    tseg = jnp.asarray(
        (np.arange(NS) * 2 * WB)[:, None, None]
        + ((np.arange(NCH * RC) // L) % (2 * WB)).reshape(NCH, RC)[None],
        dtype=jnp.int32,
    )
    cidx = category.astype(jnp.int32).reshape(NW, NCC, CCH)
    sidx = subcategory.astype(jnp.int32).reshape(NW, NCC, CCH)
    pad = ((0, 0), (0, E - CD))
    cemb_p = jnp.pad(cat_emb, pad)
    semb_p = jnp.pad(subcat_emb, pad)
    ts, cv, sv = _sc_gather(tidx, tseg, cidx, sidx, word_emb, cemb_p, semb_p)
    wpad = ((0, E - CD), (0, 0))
    return _dense(
        ts, cv, sv,
        W_title * jnp.float32(1.0 / L),
        b_title.reshape(1, CD),
        W_final[:CD],
        jnp.pad(W_final[CD:2 * CD], wpad),
        jnp.pad(W_final[2 * CD:], wpad),
        b_final.reshape(1, E),
    )
